# Initial kernel scaffold; baseline (speedup 1.0000x reference)
import jax
import jax.numpy as jnp
from jax.experimental import pallas as pl


def kernel(x, edge_attr, W, att_src, att_dst, bias, We_w, We_b, edge_index):
    raise NotImplementedError("write your pallas kernel here")


# speedup vs baseline: 27.4115x; 27.4115x over previous
import jax
import jax.numpy as jnp
from jax import lax
from jax.experimental import pallas as pl
from jax.experimental.pallas import tpu as pltpu
from jax.experimental.pallas import tpu_sc as plsc

N = 10000
D = 128
E = 320000

PADN = 10240
NW = 32
K = 64
NCHUNK = 162
EPW = K * NCHUNK
PADE = NW * EPW
STRIPE = PADN // 16
NEG = -1.0e30


def _prep_body(x_ref, w_ref, asr_ref, adr_ref, h_ref, asx_ref, adx_ref,
               g_ref):
    x = x_ref[...]
    h = lax.dot_general(x, w_ref[...], (((1,), (1,)), ((), ())),
                        precision=lax.Precision.HIGHEST,
                        preferred_element_type=jnp.float32)
    h_ref[...] = h
    sa = jnp.sum(h * asr_ref[...], axis=1, keepdims=True)
    da = jnp.sum(h * adr_ref[...], axis=1, keepdims=True)
    row = lax.broadcasted_iota(jnp.int32, (PADN, 1), 0)
    valid = row < N
    sa = jnp.where(valid, sa, NEG)
    da = jnp.where(valid, da, NEG)
    asx_ref[...] = jnp.broadcast_to(sa, (PADN, 16))
    adx_ref[...] = jnp.broadcast_to(da, (PADN, 16))
    ms = jnp.max(sa)
    md = jnp.max(da)
    m = ms + md
    g = jnp.where(m > 0, m, 0.2 * m)
    g_ref[...] = jnp.full((1, 128), g, jnp.float32)


def _prep(x_pad, w, att_src, att_dst):
    return pl.pallas_call(
        _prep_body,
        out_shape=(
            jax.ShapeDtypeStruct((PADN, D), jnp.float32),
            jax.ShapeDtypeStruct((PADN, 16), jnp.float32),
            jax.ShapeDtypeStruct((PADN, 16), jnp.float32),
            jax.ShapeDtypeStruct((1, 128), jnp.float32),
        ),
    )(x_pad, w, att_src.reshape(1, D), att_dst.reshape(1, D))


NBUF = 3


def _edge_body(h_hbm, asx_hbm, adx_hbm, si_hbm, di_hbm, g_hbm, z2_hbm, z1_hbm,
               num_hbm, den_hbm,
               rows0, rows1, rows2, sidx0, sidx1, sidx2, didx0, didx1, didx2,
               sas0, sas1, sas2, sad0, sad1, sad2,
               wbuf0, wbuf1, wbuf2, wrow0, wrow1, wrow2, gv,
               acc, den,
               sem_i0, sem_i1, sem_i2, sem_g0, sem_g1, sem_g2,
               sem_s0, sem_s1, sem_s2):
    cid = lax.axis_index("c")
    sid = lax.axis_index("s")
    wid = sid * 2 + cid

    pltpu.sync_copy(g_hbm, gv)
    pltpu.sync_copy(z2_hbm, acc.at[pl.ds(sid * STRIPE, STRIPE)])
    pltpu.sync_copy(z1_hbm, den.at[pl.ds(sid * STRIPE, STRIPE)])
    plsc.subcore_barrier()

    rows = (rows0, rows1, rows2)
    sidx = (sidx0, sidx1, sidx2)
    didx = (didx0, didx1, didx2)
    sas = (sas0, sas1, sas2)
    sad = (sad0, sad1, sad2)
    wbuf = (wbuf0, wbuf1, wbuf2)
    wrow = (wrow0, wrow1, wrow2)
    sem_i = (sem_i0, sem_i1, sem_i2)
    sem_g = (sem_g0, sem_g1, sem_g2)
    sem_s = (sem_s0, sem_s1, sem_s2)

    iota16 = lax.iota(jnp.int32, 16)
    zero16 = jnp.zeros((16,), jnp.int32)
    gvec = gv[...]
    my_si = si_hbm.at[wid]
    my_di = di_hbm.at[wid]

    def istart(c, b):
        pltpu.make_async_copy(my_si.at[pl.ds(c * K, K)], sidx[b],
                              sem_i[b]).start()
        pltpu.make_async_copy(my_di.at[pl.ds(c * K, K)], didx[b],
                              sem_i[b]).start()

    def iwait(b):
        pltpu.make_async_copy(my_si.at[pl.ds(0, K)], sidx[b], sem_i[b]).wait()
        pltpu.make_async_copy(my_di.at[pl.ds(0, K)], didx[b], sem_i[b]).wait()

    def gstart(b):
        pltpu.make_async_copy(h_hbm.at[sidx[b]], rows[b], sem_g[b]).start()
        pltpu.make_async_copy(asx_hbm.at[sidx[b]], sas[b], sem_g[b]).start()
        pltpu.make_async_copy(adx_hbm.at[didx[b]], sad[b], sem_g[b]).start()

    def gwait(b):
        pltpu.make_async_copy(h_hbm.at[sidx[b]], rows[b], sem_g[b]).wait()
        pltpu.make_async_copy(asx_hbm.at[sidx[b]], sas[b], sem_g[b]).wait()
        pltpu.make_async_copy(adx_hbm.at[didx[b]], sad[b], sem_g[b]).wait()

    def sstart(b):
        pltpu.async_copy(rows[b], acc.at[didx[b]], sem_s[b], add=True)
        pltpu.async_copy(wrow[b], den.at[didx[b]], sem_s[b], add=True)

    def swait(b):
        pltpu.make_async_copy(rows[b], acc.at[didx[b]], sem_s[b]).wait()
        pltpu.make_async_copy(wrow[b], den.at[didx[b]], sem_s[b]).wait()

    def compute(b):
        for g in range(K // 16):
            lane = g * 16 + iota16
            s = plsc.load_gather(sas[b], [lane, zero16])
            d = plsc.load_gather(sad[b], [lane, zero16])
            a = s + d
            a = jnp.where(a > 0, a, 0.2 * a)
            w = jnp.exp(a - gvec)
            wbuf[b][pl.ds(g * 16, 16)] = w

        def jbody(j, carry):
            jf = jnp.full((16,), j, jnp.int32)
            wv = plsc.load_gather(wbuf[b], [jf])
            wrow[b][j, pl.ds(0, 16)] = wv
            for t in range(8):
                rows[b][j, pl.ds(t * 16, 16)] = (
                    rows[b][j, pl.ds(t * 16, 16)] * wv)
            return carry
        lax.fori_loop(0, K, jbody, 0)

    istart(0, 0)
    istart(1, 1)
    iwait(0)
    gstart(0)
    iwait(1)
    gstart(1)

    def loop_body(i, carry):
        for b in range(NBUF):
            c = NBUF * i + b
            b2 = (b + 2) % NBUF

            @pl.when(c >= 1)
            def _():
                swait(b2)

            @pl.when(c + 2 < NCHUNK)
            def _():
                istart(c + 2, b2)
                iwait(b2)
                gstart(b2)

            gwait(b)
            compute(b)
            sstart(b)
        return carry
    lax.fori_loop(0, NCHUNK // NBUF, loop_body, 0)

    swait((NCHUNK - 1) % NBUF)

    plsc.subcore_barrier()
    r0 = sid * STRIPE
    pltpu.sync_copy(acc.at[pl.ds(r0, STRIPE)],
                    num_hbm.at[cid].at[pl.ds(r0, STRIPE)])
    pltpu.sync_copy(den.at[pl.ds(r0, STRIPE)],
                    den_hbm.at[cid].at[pl.ds(r0, STRIPE)])


def _edges(h_pad, asx, adx, si, di, gsplat, z2, z1):
    mesh = plsc.VectorSubcoreMesh(core_axis_name="c", subcore_axis_name="s")
    f = pl.kernel(
        _edge_body,
        out_type=(
            jax.ShapeDtypeStruct((2, PADN, D), jnp.float32),
            jax.ShapeDtypeStruct((2, PADN, 16), jnp.float32),
        ),
        mesh=mesh,
        scratch_types=(
            [pltpu.VMEM((K, D), jnp.float32)] * 3 +
            [pltpu.VMEM((K,), jnp.int32)] * 6 +
            [pltpu.VMEM((K, 16), jnp.float32)] * 6 +
            [pltpu.VMEM((K,), jnp.float32)] * 3 +
            [pltpu.VMEM((K, 16), jnp.float32)] * 3 +
            [pltpu.VMEM((16,), jnp.float32)] +
            [pltpu.VMEM_SHARED((PADN, D), jnp.float32),
             pltpu.VMEM_SHARED((PADN, 16), jnp.float32)] +
            [pltpu.SemaphoreType.DMA] * 9
        ),
        compiler_params=pltpu.CompilerParams(use_tc_tiling_on_sc=False,
                                             needs_layout_passes=False),
    )
    return f(h_pad, asx, adx, si, di, gsplat, z2, z1)


def _comb_body(num_ref, den_ref, b_ref, o_ref):
    num = num_ref[0] + num_ref[1]
    den = den_ref[0, :, 0:1] + den_ref[1, :, 0:1]
    o_ref[...] = num / (den + 1e-16) + b_ref[...]


def _combine(num, den, bias):
    blk = 640
    return pl.pallas_call(
        _comb_body,
        grid=(PADN // blk,),
        in_specs=[
            pl.BlockSpec((2, blk, D), lambda i: (0, i, 0)),
            pl.BlockSpec((2, blk, 16), lambda i: (0, i, 0)),
            pl.BlockSpec((1, D), lambda i: (0, 0)),
        ],
        out_specs=pl.BlockSpec((blk, D), lambda i: (i, 0)),
        out_shape=jax.ShapeDtypeStruct((PADN, D), jnp.float32),
    )(num, den, bias.reshape(1, D))


def kernel(x, edge_attr, W, att_src, att_dst, bias, We_w, We_b, edge_index):
    del edge_attr, We_w, We_b
    x_pad = jnp.pad(x, ((0, PADN - N), (0, 0)))
    loop = jnp.arange(N, dtype=jnp.int32)
    extra = PADE - E - N
    src = jnp.concatenate([edge_index[0], loop,
                           jnp.full((extra,), N, jnp.int32)])
    dst = jnp.concatenate([edge_index[1], loop,
                           jnp.zeros((extra,), jnp.int32)])
    si = src.reshape(NW, EPW)
    di = dst.reshape(NW, EPW)
    z2 = jnp.zeros((STRIPE, D), jnp.float32)
    z1 = jnp.zeros((STRIPE, 16), jnp.float32)

    h_pad, asx, adx, g = _prep(x_pad, W, att_src, att_dst)
    gsplat = g[0, :16]
    num, den = _edges(h_pad, asx, adx, si, di, gsplat, z2, z1)
    out = _combine(num, den, bias)
    return out[:N]


# speedup vs baseline: 30.6174x; 1.1170x over previous
import jax
import jax.numpy as jnp
from jax import lax
from jax.experimental import pallas as pl
from jax.experimental.pallas import tpu as pltpu
from jax.experimental.pallas import tpu_sc as plsc

N = 10000
D = 128
E = 320000

PADN = 10240
NW = 32
K = 64
NCHUNK = 162
EPW = K * NCHUNK
PADE = NW * EPW
STRIPE = PADN // 16
NEG = -1.0e30


def _prep_body(x_ref, w_ref, asr_ref, adr_ref, h_ref, asx_ref, adx_ref,
               g_ref):
    x = x_ref[...]
    h = lax.dot_general(x, w_ref[...], (((1,), (1,)), ((), ())),
                        precision=lax.Precision.HIGHEST,
                        preferred_element_type=jnp.float32)
    h_ref[...] = h
    sa = jnp.sum(h * asr_ref[...], axis=1, keepdims=True)
    da = jnp.sum(h * adr_ref[...], axis=1, keepdims=True)
    row = lax.broadcasted_iota(jnp.int32, (PADN, 1), 0)
    valid = row < N
    sa = jnp.where(valid, sa, NEG)
    da = jnp.where(valid, da, NEG)
    asx_ref[...] = jnp.broadcast_to(sa, (PADN, 16))
    adx_ref[...] = jnp.broadcast_to(da, (PADN, 16))
    ms = jnp.max(sa)
    md = jnp.max(da)
    m = ms + md
    g = jnp.where(m > 0, m, 0.2 * m)
    g_ref[...] = jnp.full((1, 128), g, jnp.float32)


def _prep(x_pad, w, att_src, att_dst):
    return pl.pallas_call(
        _prep_body,
        out_shape=(
            jax.ShapeDtypeStruct((PADN, D), jnp.float32),
            jax.ShapeDtypeStruct((PADN, 16), jnp.float32),
            jax.ShapeDtypeStruct((PADN, 16), jnp.float32),
            jax.ShapeDtypeStruct((1, 128), jnp.float32),
        ),
    )(x_pad, w, att_src.reshape(1, D), att_dst.reshape(1, D))


NBUF = 3


def _edge_body(h_hbm, asx_hbm, adx_hbm, si_hbm, di_hbm, g_hbm, z2_hbm, z1_hbm,
               num_hbm, den_hbm,
               rows0, rows1, rows2,
               sidx0, sidx1, sidx2, sidx3, sidx4, sidx5,
               didx0, didx1, didx2, didx3, didx4, didx5,
               sas0, sas1, sas2, sad0, sad1, sad2,
               wbuf0, wbuf1, wbuf2, wrow0, wrow1, wrow2, gv,
               acc, den,
               sem_i0, sem_i1, sem_i2, sem_i3, sem_i4, sem_i5,
               sem_g0, sem_g1, sem_g2,
               sem_s0, sem_s1, sem_s2):
    cid = lax.axis_index("c")
    sid = lax.axis_index("s")
    wid = sid * 2 + cid

    pltpu.sync_copy(g_hbm, gv)
    pltpu.sync_copy(z2_hbm, acc.at[pl.ds(sid * STRIPE, STRIPE)])
    pltpu.sync_copy(z1_hbm, den.at[pl.ds(sid * STRIPE, STRIPE)])
    plsc.subcore_barrier()

    rows = (rows0, rows1, rows2)
    sidx = (sidx0, sidx1, sidx2, sidx3, sidx4, sidx5)
    didx = (didx0, didx1, didx2, didx3, didx4, didx5)
    sas = (sas0, sas1, sas2)
    sad = (sad0, sad1, sad2)
    wbuf = (wbuf0, wbuf1, wbuf2)
    wrow = (wrow0, wrow1, wrow2)
    sem_i = (sem_i0, sem_i1, sem_i2, sem_i3, sem_i4, sem_i5)
    sem_g = (sem_g0, sem_g1, sem_g2)
    sem_s = (sem_s0, sem_s1, sem_s2)

    iota16 = lax.iota(jnp.int32, 16)
    zero16 = jnp.zeros((16,), jnp.int32)
    gvec = gv[...]
    my_si = si_hbm.at[wid]
    my_di = di_hbm.at[wid]

    def istart(c, bi):
        pltpu.make_async_copy(my_si.at[pl.ds(c * K, K)], sidx[bi],
                              sem_i[bi]).start()
        pltpu.make_async_copy(my_di.at[pl.ds(c * K, K)], didx[bi],
                              sem_i[bi]).start()

    def iwait(bi):
        pltpu.make_async_copy(my_si.at[pl.ds(0, K)], sidx[bi],
                              sem_i[bi]).wait()
        pltpu.make_async_copy(my_di.at[pl.ds(0, K)], didx[bi],
                              sem_i[bi]).wait()

    def gstart(b, bi):
        pltpu.make_async_copy(h_hbm.at[sidx[bi]], rows[b], sem_g[b]).start()
        pltpu.make_async_copy(asx_hbm.at[sidx[bi]], sas[b], sem_g[b]).start()
        pltpu.make_async_copy(adx_hbm.at[didx[bi]], sad[b], sem_g[b]).start()

    def gwait(b, bi):
        pltpu.make_async_copy(h_hbm.at[sidx[bi]], rows[b], sem_g[b]).wait()
        pltpu.make_async_copy(asx_hbm.at[sidx[bi]], sas[b], sem_g[b]).wait()
        pltpu.make_async_copy(adx_hbm.at[didx[bi]], sad[b], sem_g[b]).wait()

    def sstart(b, bi):
        pltpu.async_copy(rows[b], acc.at[didx[bi]], sem_s[b], add=True)
        pltpu.async_copy(wrow[b], den.at[didx[bi]], sem_s[b], add=True)

    def swait(b, bi):
        pltpu.make_async_copy(rows[b], acc.at[didx[bi]], sem_s[b]).wait()
        pltpu.make_async_copy(wrow[b], den.at[didx[bi]], sem_s[b]).wait()

    def compute(b):
        for g in range(K // 16):
            lane = g * 16 + iota16
            s = plsc.load_gather(sas[b], [lane, zero16])
            d = plsc.load_gather(sad[b], [lane, zero16])
            a = s + d
            a = jnp.where(a > 0, a, 0.2 * a)
            w = jnp.exp(a - gvec)
            wbuf[b][pl.ds(g * 16, 16)] = w

        def jbody(jj, carry):
            for u in range(2):
                j = 2 * jj + u
                jf = jnp.full((16,), j, jnp.int32)
                wv = plsc.load_gather(wbuf[b], [jf])
                wrow[b][j, pl.ds(0, 16)] = wv
                for t in range(8):
                    rows[b][j, pl.ds(t * 16, 16)] = (
                        rows[b][j, pl.ds(t * 16, 16)] * wv)
            return carry
        lax.fori_loop(0, K // 2, jbody, 0)

    istart(0, 0)
    istart(1, 1)
    istart(2, 2)
    iwait(0)
    gstart(0, 0)
    iwait(1)
    gstart(1, 1)

    def loop_body(i, carry):
        for k in range(2 * NBUF):
            c = 2 * NBUF * i + k
            b = k % NBUF

            @pl.when(c >= 1)
            def _():
                swait((b + 2) % NBUF, (k + 5) % (2 * NBUF))

            @pl.when(c + 2 < NCHUNK)
            def _():
                iwait((k + 2) % (2 * NBUF))
                gstart((b + 2) % NBUF, (k + 2) % (2 * NBUF))

            @pl.when(c + 3 < NCHUNK)
            def _():
                istart(c + 3, (k + 3) % (2 * NBUF))

            gwait(b, k)
            compute(b)
            sstart(b, k)
        return carry
    lax.fori_loop(0, NCHUNK // (2 * NBUF), loop_body, 0)

    swait((NCHUNK - 1) % NBUF, (NCHUNK - 1) % (2 * NBUF))

    plsc.subcore_barrier()
    r0 = sid * STRIPE
    pltpu.sync_copy(acc.at[pl.ds(r0, STRIPE)],
                    num_hbm.at[cid].at[pl.ds(r0, STRIPE)])
    pltpu.sync_copy(den.at[pl.ds(r0, STRIPE)],
                    den_hbm.at[cid].at[pl.ds(r0, STRIPE)])


def _edges(h_pad, asx, adx, si, di, gsplat, z2, z1):
    mesh = plsc.VectorSubcoreMesh(core_axis_name="c", subcore_axis_name="s")
    f = pl.kernel(
        _edge_body,
        out_type=(
            jax.ShapeDtypeStruct((2, PADN, D), jnp.float32),
            jax.ShapeDtypeStruct((2, PADN, 16), jnp.float32),
        ),
        mesh=mesh,
        scratch_types=(
            [pltpu.VMEM((K, D), jnp.float32)] * 3 +
            [pltpu.VMEM((K,), jnp.int32)] * 12 +
            [pltpu.VMEM((K, 16), jnp.float32)] * 6 +
            [pltpu.VMEM((K,), jnp.float32)] * 3 +
            [pltpu.VMEM((K, 16), jnp.float32)] * 3 +
            [pltpu.VMEM((16,), jnp.float32)] +
            [pltpu.VMEM_SHARED((PADN, D), jnp.float32),
             pltpu.VMEM_SHARED((PADN, 16), jnp.float32)] +
            [pltpu.SemaphoreType.DMA] * 12
        ),
        compiler_params=pltpu.CompilerParams(use_tc_tiling_on_sc=False,
                                             needs_layout_passes=False),
    )
    return f(h_pad, asx, adx, si, di, gsplat, z2, z1)


def _comb_body(num_ref, den_ref, b_ref, o_ref):
    num = num_ref[0] + num_ref[1]
    den = den_ref[0, :, 0:1] + den_ref[1, :, 0:1]
    o_ref[...] = num / (den + 1e-16) + b_ref[...]


def _combine(num, den, bias):
    blk = 640
    return pl.pallas_call(
        _comb_body,
        grid=(PADN // blk,),
        in_specs=[
            pl.BlockSpec((2, blk, D), lambda i: (0, i, 0)),
            pl.BlockSpec((2, blk, 16), lambda i: (0, i, 0)),
            pl.BlockSpec((1, D), lambda i: (0, 0)),
        ],
        out_specs=pl.BlockSpec((blk, D), lambda i: (i, 0)),
        out_shape=jax.ShapeDtypeStruct((PADN, D), jnp.float32),
    )(num, den, bias.reshape(1, D))


def kernel(x, edge_attr, W, att_src, att_dst, bias, We_w, We_b, edge_index):
    del edge_attr, We_w, We_b
    x_pad = jnp.pad(x, ((0, PADN - N), (0, 0)))
    loop = jnp.arange(N, dtype=jnp.int32)
    extra = PADE - E - N
    src = jnp.concatenate([edge_index[0], loop,
                           jnp.full((extra,), N, jnp.int32)])
    dst = jnp.concatenate([edge_index[1], loop,
                           jnp.zeros((extra,), jnp.int32)])
    si = src.reshape(NW, EPW)
    di = dst.reshape(NW, EPW)
    z2 = jnp.zeros((STRIPE, D), jnp.float32)
    z1 = jnp.zeros((STRIPE, 16), jnp.float32)

    h_pad, asx, adx, g = _prep(x_pad, W, att_src, att_dst)
    gsplat = g[0, :16]
    num, den = _edges(h_pad, asx, adx, si, di, gsplat, z2, z1)
    out = _combine(num, den, bias)
    return out[:N]


# speedup vs baseline: 34.7039x; 1.1335x over previous
import jax
import jax.numpy as jnp
from jax import lax
from jax.experimental import pallas as pl
from jax.experimental.pallas import tpu as pltpu
from jax.experimental.pallas import tpu_sc as plsc

N = 10000
D = 128
E = 320000

PADN = 10240
NW = 32
K = 64
NCHUNK = 162
EPW = K * NCHUNK
PADE = NW * EPW
STRIPE = PADN // 16
NEG = -1.0e30


def _prep_body(x_ref, w_ref, asr_ref, adr_ref, h_ref, asx_ref, adx_ref,
               g_ref):
    x = x_ref[...]
    h = lax.dot_general(x, w_ref[...], (((1,), (1,)), ((), ())),
                        precision=lax.Precision.HIGHEST,
                        preferred_element_type=jnp.float32)
    h_ref[...] = h
    sa = jnp.sum(h * asr_ref[...], axis=1, keepdims=True)
    da = jnp.sum(h * adr_ref[...], axis=1, keepdims=True)
    row = lax.broadcasted_iota(jnp.int32, (PADN, 1), 0)
    valid = row < N
    sa = jnp.where(valid, sa, NEG)
    da = jnp.where(valid, da, NEG)
    asx_ref[...] = jnp.broadcast_to(sa, (PADN, 16))
    adx_ref[...] = jnp.broadcast_to(da, (PADN, 16))
    ms = jnp.max(sa)
    md = jnp.max(da)
    m = ms + md
    g = jnp.where(m > 0, m, 0.2 * m)
    g_ref[...] = jnp.full((1, 128), g, jnp.float32)


def _prep(x_pad, w, att_src, att_dst):
    return pl.pallas_call(
        _prep_body,
        out_shape=(
            jax.ShapeDtypeStruct((PADN, D), jnp.float32),
            jax.ShapeDtypeStruct((PADN, 16), jnp.float32),
            jax.ShapeDtypeStruct((PADN, 16), jnp.float32),
            jax.ShapeDtypeStruct((1, 128), jnp.float32),
        ),
    )(x_pad, w, att_src.reshape(1, D), att_dst.reshape(1, D))


NBUF = 3


def _edge_body(h_hbm, asx_hbm, adx_hbm, si_hbm, di_hbm, g_hbm, z2_hbm, z1_hbm,
               num_hbm, den_hbm,
               rows0, rows1, rows2,
               sidx0, sidx1, sidx2, sidx3, sidx4, sidx5,
               didx0, didx1, didx2, didx3, didx4, didx5,
               sas0, sas1, sas2, sad0, sad1, sad2,
               wbuf0, wbuf1, wbuf2, wrow0, wrow1, wrow2, gv,
               acc, den,
               sem_i0, sem_i1, sem_i2, sem_i3, sem_i4, sem_i5,
               sem_g0, sem_g1, sem_g2,
               sem_s0, sem_s1, sem_s2):
    cid = lax.axis_index("c")
    sid = lax.axis_index("s")
    wid = sid * 2 + cid

    pltpu.sync_copy(g_hbm, gv)
    pltpu.sync_copy(z2_hbm, acc.at[pl.ds(sid * STRIPE, STRIPE)])
    pltpu.sync_copy(z1_hbm, den.at[pl.ds(sid * STRIPE, STRIPE)])
    plsc.subcore_barrier()

    rows = (rows0, rows1, rows2)
    sidx = (sidx0, sidx1, sidx2, sidx3, sidx4, sidx5)
    didx = (didx0, didx1, didx2, didx3, didx4, didx5)
    sas = (sas0, sas1, sas2)
    sad = (sad0, sad1, sad2)
    wbuf = (wbuf0, wbuf1, wbuf2)
    wrow = (wrow0, wrow1, wrow2)
    sem_i = (sem_i0, sem_i1, sem_i2, sem_i3, sem_i4, sem_i5)
    sem_g = (sem_g0, sem_g1, sem_g2)
    sem_s = (sem_s0, sem_s1, sem_s2)

    iota16 = lax.iota(jnp.int32, 16)
    zero16 = jnp.zeros((16,), jnp.int32)
    gvec = gv[...]
    my_si = si_hbm.at[wid]
    my_di = di_hbm.at[wid]

    def istart(c, bi):
        pltpu.make_async_copy(my_si.at[pl.ds(c * K, K)], sidx[bi],
                              sem_i[bi]).start()
        pltpu.make_async_copy(my_di.at[pl.ds(c * K, K)], didx[bi],
                              sem_i[bi]).start()

    def iwait(bi):
        pltpu.make_async_copy(my_si.at[pl.ds(0, K)], sidx[bi],
                              sem_i[bi]).wait()
        pltpu.make_async_copy(my_di.at[pl.ds(0, K)], didx[bi],
                              sem_i[bi]).wait()

    def gstart(b, bi):
        pltpu.make_async_copy(h_hbm.at[sidx[bi]], rows[b], sem_g[b]).start()
        pltpu.make_async_copy(asx_hbm.at[sidx[bi]], sas[b], sem_g[b]).start()
        pltpu.make_async_copy(adx_hbm.at[didx[bi]], sad[b], sem_g[b]).start()

    def gwait(b, bi):
        pltpu.make_async_copy(h_hbm.at[sidx[bi]], rows[b], sem_g[b]).wait()
        pltpu.make_async_copy(asx_hbm.at[sidx[bi]], sas[b], sem_g[b]).wait()
        pltpu.make_async_copy(adx_hbm.at[didx[bi]], sad[b], sem_g[b]).wait()

    def sstart(b, bi):
        pltpu.async_copy(rows[b], acc.at[didx[bi]], sem_s[b], add=True)
        pltpu.async_copy(wrow[b], den.at[didx[bi]], sem_s[b], add=True)

    def swait(b, bi):
        pltpu.make_async_copy(rows[b], acc.at[didx[bi]], sem_s[b]).wait()
        pltpu.make_async_copy(wrow[b], den.at[didx[bi]], sem_s[b]).wait()

    def compute(b):
        for g in range(K // 16):
            lane = g * 16 + iota16
            s = plsc.load_gather(sas[b], [lane, zero16])
            d = plsc.load_gather(sad[b], [lane, zero16])
            a = s + d
            a = jnp.where(a > 0, a, 0.2 * a)
            w = jnp.exp(a - gvec)
            wbuf[b][pl.ds(g * 16, 16)] = w

        def jbody(jj, carry):
            base = 4 * jj
            wvs = []
            for u in range(4):
                jf = jnp.full((16,), base + u, jnp.int32)
                wvs.append(plsc.load_gather(wbuf[b], [jf]))
            for u in range(4):
                wrow[b][base + u, pl.ds(0, 16)] = wvs[u]
            for u in range(4):
                j = base + u
                for t in range(8):
                    rows[b][j, pl.ds(t * 16, 16)] = (
                        rows[b][j, pl.ds(t * 16, 16)] * wvs[u])
            return carry
        lax.fori_loop(0, K // 4, jbody, 0)

    istart(0, 0)
    istart(1, 1)
    istart(2, 2)
    iwait(0)
    gstart(0, 0)
    iwait(1)
    gstart(1, 1)

    def loop_body(i, carry):
        for k in range(2 * NBUF):
            c = 2 * NBUF * i + k
            b = k % NBUF

            @pl.when(c >= 1)
            def _():
                swait((b + 2) % NBUF, (k + 5) % (2 * NBUF))

            @pl.when(c + 2 < NCHUNK)
            def _():
                iwait((k + 2) % (2 * NBUF))
                gstart((b + 2) % NBUF, (k + 2) % (2 * NBUF))

            @pl.when(c + 3 < NCHUNK)
            def _():
                istart(c + 3, (k + 3) % (2 * NBUF))

            gwait(b, k)
            compute(b)
            sstart(b, k)
        return carry
    lax.fori_loop(0, NCHUNK // (2 * NBUF), loop_body, 0)

    swait((NCHUNK - 1) % NBUF, (NCHUNK - 1) % (2 * NBUF))

    plsc.subcore_barrier()
    r0 = sid * STRIPE
    pltpu.sync_copy(acc.at[pl.ds(r0, STRIPE)],
                    num_hbm.at[cid].at[pl.ds(r0, STRIPE)])
    pltpu.sync_copy(den.at[pl.ds(r0, STRIPE)],
                    den_hbm.at[cid].at[pl.ds(r0, STRIPE)])


def _edges(h_pad, asx, adx, si, di, gsplat, z2, z1):
    mesh = plsc.VectorSubcoreMesh(core_axis_name="c", subcore_axis_name="s")
    f = pl.kernel(
        _edge_body,
        out_type=(
            jax.ShapeDtypeStruct((2, PADN, D), jnp.float32),
            jax.ShapeDtypeStruct((2, PADN, 16), jnp.float32),
        ),
        mesh=mesh,
        scratch_types=(
            [pltpu.VMEM((K, D), jnp.float32)] * 3 +
            [pltpu.VMEM((K,), jnp.int32)] * 12 +
            [pltpu.VMEM((K, 16), jnp.float32)] * 6 +
            [pltpu.VMEM((K,), jnp.float32)] * 3 +
            [pltpu.VMEM((K, 16), jnp.float32)] * 3 +
            [pltpu.VMEM((16,), jnp.float32)] +
            [pltpu.VMEM_SHARED((PADN, D), jnp.float32),
             pltpu.VMEM_SHARED((PADN, 16), jnp.float32)] +
            [pltpu.SemaphoreType.DMA] * 12
        ),
        compiler_params=pltpu.CompilerParams(use_tc_tiling_on_sc=False,
                                             needs_layout_passes=False),
    )
    return f(h_pad, asx, adx, si, di, gsplat, z2, z1)


def _comb_body(num_ref, den_ref, b_ref, o_ref):
    num = num_ref[0] + num_ref[1]
    den = den_ref[0, :, 0:1] + den_ref[1, :, 0:1]
    o_ref[...] = num / (den + 1e-16) + b_ref[...]


def _combine(num, den, bias):
    blk = 640
    return pl.pallas_call(
        _comb_body,
        grid=(PADN // blk,),
        in_specs=[
            pl.BlockSpec((2, blk, D), lambda i: (0, i, 0)),
            pl.BlockSpec((2, blk, 16), lambda i: (0, i, 0)),
            pl.BlockSpec((1, D), lambda i: (0, 0)),
        ],
        out_specs=pl.BlockSpec((blk, D), lambda i: (i, 0)),
        out_shape=jax.ShapeDtypeStruct((PADN, D), jnp.float32),
    )(num, den, bias.reshape(1, D))


def kernel(x, edge_attr, W, att_src, att_dst, bias, We_w, We_b, edge_index):
    del edge_attr, We_w, We_b
    x_pad = jnp.pad(x, ((0, PADN - N), (0, 0)))
    loop = jnp.arange(N, dtype=jnp.int32)
    extra = PADE - E - N
    src = jnp.concatenate([edge_index[0], loop,
                           jnp.full((extra,), N, jnp.int32)])
    dst = jnp.concatenate([edge_index[1], loop,
                           jnp.zeros((extra,), jnp.int32)])
    si = src.reshape(NW, EPW)
    di = dst.reshape(NW, EPW)
    z2 = jnp.zeros((STRIPE, D), jnp.float32)
    z1 = jnp.zeros((STRIPE, 16), jnp.float32)

    h_pad, asx, adx, g = _prep(x_pad, W, att_src, att_dst)
    gsplat = g[0, :16]
    num, den = _edges(h_pad, asx, adx, si, di, gsplat, z2, z1)
    out = _combine(num, den, bias)
    return out[:N]
